# baseline (speedup 1.0000x reference)
import functools

import jax
import jax.numpy as jnp
from jax import lax
from jax.experimental import pallas as pl
from jax.experimental.pallas import tpu as pltpu
from jax.experimental.pallas import tpu_sc as plsc

_U = 10000
_I = 10000
_FDIM = 128
_NNZ = 320000
_NPREF = 32
_B = 4096

_NTILES = 32
_FPT = _FDIM // _NTILES
_NPAIR = _FPT // 2
_SLICE = _FPT * _U
_PSLICE = _NPAIR * _U
_OUTW = _FPT * _B
_EC = 1600
_NCH = _NNZ // _EC
_VPC = _EC // 16

_mesh = plsc.VectorSubcoreMesh(core_axis_name="c", subcore_axis_name="s")
_gout = jax.ShapeDtypeStruct((_NTILES * _OUTW,), jnp.float32)


@functools.partial(
    pl.kernel,
    out_type=tuple([_gout] * 23),
    mesh=_mesh,
    scratch_types=[
        pltpu.VMEM((_PSLICE,), jnp.int32),
        pltpu.VMEM((_SLICE,), jnp.float32),
        pltpu.VMEM((_OUTW,), jnp.float32),
        pltpu.VMEM((_B,), jnp.int32),
        pltpu.VMEM((_B,), jnp.int32),
        pltpu.VMEM((_B,), jnp.int32),
        pltpu.VMEM((_B,), jnp.int32),
        pltpu.VMEM((_B,), jnp.int32),
        pltpu.VMEM((_EC,), jnp.int32),
        pltpu.VMEM((_EC,), jnp.int32),
        pltpu.VMEM((_EC,), jnp.float32),
        pltpu.VMEM((_EC,), jnp.float32),
        pltpu.SemaphoreType.DMA,
        pltpu.SemaphoreType.DMA,
    ],
    compiler_params=pltpu.CompilerParams(needs_layout_passes=False),
)
def _sc_gcn(IembT, UembT, IembP, UembP, oldUT, oldIT, oldU1T, pA, vA, pB, vB,
            user, item_i, item_j, u_sample, i_sample,
            o_Ie_ii, o_Ie_ij, o_Ie_is,
            o_g1u_user, o_g1u_us,
            o_g2i_ii, o_g2i_ij, o_g2i_is,
            o_g3u_user, o_g3u_us,
            o_Ue_user, o_Ue_us,
            o_g1i_ii, o_g1i_ij, o_g1i_is,
            o_g2u_user, o_g2u_us,
            o_g3i_ii, o_g3i_ij, o_g3i_is,
            o_oU_us, o_oI_is, o_oU1_us,
            pk, acc, tmp, x_user, x_us, x_ii, x_ij, x_is,
            ep0, ep1, ev0, ev1, s0, s1):
    w = lax.axis_index("s") * 2 + lax.axis_index("c")
    tb = w * _SLICE
    pbase = w * _PSLICE
    to = w * _OUTW

    pltpu.sync_copy(user, x_user)
    pltpu.sync_copy(u_sample, x_us)
    pltpu.sync_copy(item_i, x_ii)
    pltpu.sync_copy(item_j, x_ij)
    pltpu.sync_copy(i_sample, x_is)

    aslc = [acc.at[pl.ds(f * _U, _U)] for f in range(_FPT)]
    pslc = [pk.at[pl.ds(j * _U, _U)] for j in range(_NPAIR)]

    def load_f32_slice(src_hbm):
        pltpu.sync_copy(src_hbm.at[pl.ds(tb, _SLICE)], acc)

    def load_packed_slice(src_hbm):
        pltpu.sync_copy(src_hbm.at[pl.ds(pbase, _PSLICE)], pk)

    def zero_acc():
        z = jnp.zeros((16,), jnp.float32)

        @plsc.parallel_loop(0, _SLICE // 16, unroll=8)
        def _(i):
            acc[pl.ds(i * 16, 16)] = z

    def repack():
        @plsc.parallel_loop(0, _U // 16, unroll=4)
        def _(i):
            for j in range(_NPAIR):
                a = acc[pl.ds((2 * j) * _U + i * 16, 16)]
                b = acc[pl.ds((2 * j + 1) * _U + i * 16, 16)]
                p = plsc.pack(a, b, format=plsc.PackFormat.INTERLEAVED)
                pk[pl.ds(j * _U + i * 16, 16)] = plsc.bitcast(p, jnp.int32)

    def gather_dump(idxref, out_hbm):
        @plsc.parallel_loop(0, _B // 16, unroll=4)
        def _(i):
            ix = idxref[pl.ds(i * 16, 16)]
            for f in range(_FPT):
                g = plsc.load_gather(aslc[f], [ix])
                tmp[pl.ds(f * _B + i * 16, 16)] = g

        pltpu.sync_copy(tmp, out_hbm.at[pl.ds(to, _OUTW)])

    def edge_pass(phbm, vhbm):
        pltpu.async_copy(phbm.at[pl.ds(0, _EC)], ep0, s0)
        pltpu.async_copy(vhbm.at[pl.ds(0, _EC)], ev0, s0)
        zero_acc()

        def consume(epb, evb):
            @plsc.parallel_loop(0, _VPC, unroll=10)
            def _(i):
                pkv = epb[pl.ds(i * 16, 16)]
                vv = evb[pl.ds(i * 16, 16)]
                src = jnp.bitwise_and(pkv, 0x3FFF)
                dst = jnp.right_shift(pkv, 14)
                for j in range(_NPAIR):
                    gw = plsc.load_gather(pslc[j], [src])
                    ga, gb = plsc.unpack(
                        plsc.bitcast(gw, jnp.bfloat16),
                        format=plsc.PackFormat.INTERLEAVED)
                    plsc.addupdate_scatter(aslc[2 * j], [dst], ga * vv)
                    plsc.addupdate_scatter(aslc[2 * j + 1], [dst], gb * vv)

        def drain(epb, evb, sem):
            pltpu.make_async_copy(phbm.at[pl.ds(0, _EC)], epb, sem).wait()
            pltpu.make_async_copy(vhbm.at[pl.ds(0, _EC)], evb, sem).wait()

        def outer(k, carry):
            c1 = 2 * k + 1
            pltpu.async_copy(phbm.at[pl.ds(c1 * _EC, _EC)], ep1, s1)
            pltpu.async_copy(vhbm.at[pl.ds(c1 * _EC, _EC)], ev1, s1)
            drain(ep0, ev0, s0)
            consume(ep0, ev0)

            @pl.when(k < _NCH // 2 - 1)
            def _():
                c2 = 2 * k + 2
                pltpu.async_copy(phbm.at[pl.ds(c2 * _EC, _EC)], ep0, s0)
                pltpu.async_copy(vhbm.at[pl.ds(c2 * _EC, _EC)], ev0, s0)

            drain(ep1, ev1, s1)
            consume(ep1, ev1)
            return carry

        lax.fori_loop(0, _NCH // 2, outer, 0)

    load_f32_slice(IembT)
    gather_dump(x_ii, o_Ie_ii)
    gather_dump(x_ij, o_Ie_ij)
    gather_dump(x_is, o_Ie_is)
    load_packed_slice(IembP)

    edge_pass(pA, vA)
    gather_dump(x_user, o_g1u_user)
    gather_dump(x_us, o_g1u_us)
    repack()

    edge_pass(pB, vB)
    gather_dump(x_ii, o_g2i_ii)
    gather_dump(x_ij, o_g2i_ij)
    gather_dump(x_is, o_g2i_is)
    repack()

    edge_pass(pA, vA)
    gather_dump(x_user, o_g3u_user)
    gather_dump(x_us, o_g3u_us)

    load_f32_slice(UembT)
    gather_dump(x_user, o_Ue_user)
    gather_dump(x_us, o_Ue_us)
    load_packed_slice(UembP)

    edge_pass(pB, vB)
    gather_dump(x_ii, o_g1i_ii)
    gather_dump(x_ij, o_g1i_ij)
    gather_dump(x_is, o_g1i_is)
    repack()

    edge_pass(pA, vA)
    gather_dump(x_user, o_g2u_user)
    gather_dump(x_us, o_g2u_us)
    repack()

    edge_pass(pB, vB)
    gather_dump(x_ii, o_g3i_ii)
    gather_dump(x_ij, o_g3i_ij)
    gather_dump(x_is, o_g3i_is)

    load_f32_slice(oldUT)
    gather_dump(x_us, o_oU_us)
    load_f32_slice(oldIT)
    gather_dump(x_is, o_oI_is)
    load_f32_slice(oldU1T)
    gather_dump(x_us, o_oU1_us)


_BT = 512
_NSTEP = _B // _BT


def _softplus(x):
    return jnp.maximum(x, 0.0) + jnp.log(1.0 + jnp.exp(-jnp.abs(x)))


def _leaky(x):
    return jnp.where(x >= 0.0, x, 0.01 * x)


def _tc_body(Ie_ii, Ie_ij, Ie_is, g1u_user, g1u_us, g2i_ii, g2i_ij, g2i_is,
             g3u_user, g3u_us, Ue_user, Ue_us, g1i_ii, g1i_ij, g1i_is,
             g2u_user, g2u_us, g3i_ii, g3i_ij, g3i_is, oU_us, oI_is, oU1_us,
             pw0, pb0, pw1, pb1, pw2, pb2, sw0, sb0, sw1t, sb1,
             out, acc):
    step = pl.program_id(0)

    @pl.when(step == 0)
    def _():
        acc[0] = 0.0
        acc[1] = 0.0
        acc[2] = 0.0
        acc[3] = 0.0

    c1, c2, c3 = 0.5, 1.0 / 3.0, 0.25
    u_user = Ue_user[...] + c1 * g1u_user[...] + c2 * g2u_user[...] + c3 * g3u_user[...]
    u_us = Ue_us[...] + c1 * g1u_us[...] + c2 * g2u_us[...] + c3 * g3u_us[...]
    i_ii = Ie_ii[...] + c1 * g1i_ii[...] + c2 * g2i_ii[...] + c3 * g3i_ii[...]
    i_ij = Ie_ij[...] + c1 * g1i_ij[...] + c2 * g2i_ij[...] + c3 * g3i_ij[...]
    i_is = Ie_is[...] + c1 * g1i_is[...] + c2 * g2i_is[...] + c3 * g3i_is[...]

    pred_i = jnp.sum(u_user * i_ii, axis=0, keepdims=True)
    pred_j = jnp.sum(u_user * i_ij, axis=0, keepdims=True)
    s_bpr = jnp.sum(_softplus(pred_j - pred_i))
    s_reg = jnp.sum(u_user * u_user + i_ii * i_ii + i_ij * i_ij)

    def mlp_pref(xT):
        dn = (((0,), (0,)), ((), ()))
        h = _leaky(lax.dot_general(pw0[...], xT, dn,
                                   preferred_element_type=jnp.float32) + pb0[...])
        h = _leaky(lax.dot_general(pw1[...], h, dn,
                                   preferred_element_type=jnp.float32) + pb1[...])
        p = lax.dot_general(pw2[...], h, dn,
                            preferred_element_type=jnp.float32) + pb2[...]
        p = p - jnp.max(p, axis=0, keepdims=True)
        e = jnp.exp(p)
        return e / jnp.sum(e, axis=0, keepdims=True)

    p_new = mlp_pref(g1u_us[...])
    p_old = mlp_pref(oU1_us[...])
    s = (p_old - p_new) * (p_old - p_new)
    dn = (((0,), (0,)), ((), ()))
    hs = jnp.maximum(lax.dot_general(sw0[...], s, dn,
                                     preferred_element_type=jnp.float32) + sb0[...],
                     0.0)
    wu = _softplus(lax.dot_general(sw1t[...], hs, (((1,), (0,)), ((), ())),
                                   preferred_element_type=jnp.float32) + sb1[...])
    s_wu = jnp.sum(wu)

    diff = (jnp.sum(u_us * i_is, axis=0, keepdims=True)
            - jnp.sum(oU_us[...] * oI_is[...], axis=0, keepdims=True))
    s_d2 = jnp.sum(diff * diff)

    acc[0] += s_bpr
    acc[1] += s_reg
    acc[2] += s_wu
    acc[3] += s_d2

    @pl.when(step == _NSTEP - 1)
    def _():
        inv_b = 1.0 / _B
        loss_bpr = acc[0] * inv_b + 1e-4 * acc[1] * inv_b
        loss_kd = (acc[2] * inv_b) * (acc[3] * inv_b)
        out[0, 0] = loss_bpr + 0.01 * loss_kd


def _tc_loss(gathered, pw0, pb0, pw1, pb1, pw2, pb2, sw0, sb0, sw1, sb1):
    col = pl.BlockSpec((_FDIM, _BT), lambda i: (0, i))

    def full(shape):
        return pl.BlockSpec(shape, lambda i: tuple(0 for _ in shape))

    in_specs = [col] * 23 + [
        full((_FDIM, _FDIM)), full((_FDIM, 1)),
        full((_FDIM, _FDIM)), full((_FDIM, 1)),
        full((_FDIM, _NPREF)), full((_NPREF, 1)),
        full((_NPREF, _FDIM)), full((_FDIM, 1)),
        full((1, _FDIM)), full((1, 1)),
    ]
    out = pl.pallas_call(
        _tc_body,
        grid=(_NSTEP,),
        in_specs=in_specs,
        out_specs=pl.BlockSpec(memory_space=pltpu.SMEM),
        out_shape=jax.ShapeDtypeStruct((1, 1), jnp.float32),
        scratch_shapes=[pltpu.SMEM((4,), jnp.float32)],
        compiler_params=pltpu.CompilerParams(
            dimension_semantics=("arbitrary",)),
    )(*gathered,
      pw0, pb0.reshape(_FDIM, 1), pw1, pb1.reshape(_FDIM, 1),
      pw2, pb2.reshape(_NPREF, 1), sw0, sb0.reshape(_FDIM, 1),
      sw1.reshape(1, _FDIM), sb1.reshape(1, 1))
    return out[0, 0]


def _pack_pairs(tT):
    pairs = tT.reshape(_FDIM // 2, 2, -1)
    lo = lax.bitcast_convert_type(
        pairs[:, 0, :].astype(jnp.bfloat16), jnp.uint16).astype(jnp.uint32)
    hi = lax.bitcast_convert_type(
        pairs[:, 1, :].astype(jnp.bfloat16), jnp.uint16).astype(jnp.uint32)
    return lax.bitcast_convert_type(lo | (hi << 16), jnp.int32).reshape(-1)


def kernel(user, item_i, item_j, ui_rows, ui_cols, ui_vals, iu_vals,
           embed_user_weight, embed_item_weight, u_sample, i_sample,
           old_U_emb, old_I_emb, old_User1,
           pw0, pb0, pw1, pb1, pw2, pb2, sw0, sb0, sw1, sb1):
    IembT = embed_item_weight.T
    UembT = embed_user_weight.T
    oldUT = old_U_emb.T.reshape(-1)
    oldIT = old_I_emb.T.reshape(-1)
    oldU1T = old_User1.T.reshape(-1)
    rows = ui_rows.astype(jnp.int32)
    cols = ui_cols.astype(jnp.int32)
    pA = jnp.left_shift(rows, 14) | cols
    pB = jnp.left_shift(cols, 14) | rows

    gathered = _sc_gcn(IembT.reshape(-1), UembT.reshape(-1),
                       _pack_pairs(IembT), _pack_pairs(UembT),
                       oldUT, oldIT, oldU1T,
                       pA, ui_vals, pB, iu_vals,
                       user.astype(jnp.int32), item_i.astype(jnp.int32),
                       item_j.astype(jnp.int32), u_sample.astype(jnp.int32),
                       i_sample.astype(jnp.int32))
    gathered = [g.reshape(_FDIM, _B) for g in gathered]
    return _tc_loss(gathered, pw0, pb0, pw1, pb1, pw2, pb2,
                    sw0, sb0, sw1, sb1)


# speedup vs baseline: 1.0083x; 1.0083x over previous
import functools

import jax
import jax.numpy as jnp
from jax import lax
from jax.experimental import pallas as pl
from jax.experimental.pallas import tpu as pltpu
from jax.experimental.pallas import tpu_sc as plsc

_U = 10000
_I = 10000
_FDIM = 128
_NNZ = 320000
_NPREF = 32
_B = 4096

_NTILES = 32
_FPT = _FDIM // _NTILES
_NPAIR = _FPT // 2
_SLICE = _FPT * _U
_PSLICE = _NPAIR * _U
_OUTW = _FPT * _B
_EC = 2000
_NCH = _NNZ // _EC
_VPC = _EC // 16

_mesh = plsc.VectorSubcoreMesh(core_axis_name="c", subcore_axis_name="s")
_gout = jax.ShapeDtypeStruct((_NTILES * _OUTW,), jnp.float32)


@functools.partial(
    pl.kernel,
    out_type=tuple([_gout] * 23),
    mesh=_mesh,
    scratch_types=[
        pltpu.VMEM((_PSLICE,), jnp.int32),
        pltpu.VMEM((_SLICE,), jnp.float32),
        pltpu.VMEM((_OUTW,), jnp.float32),
        pltpu.VMEM((_B,), jnp.int32),
        pltpu.VMEM((_B,), jnp.int32),
        pltpu.VMEM((_B,), jnp.int32),
        pltpu.VMEM((_B,), jnp.int32),
        pltpu.VMEM((_B,), jnp.int32),
        pltpu.VMEM((_EC,), jnp.int32),
        pltpu.VMEM((_EC,), jnp.int32),
        pltpu.VMEM((_EC,), jnp.float32),
        pltpu.VMEM((_EC,), jnp.float32),
        pltpu.SemaphoreType.DMA,
        pltpu.SemaphoreType.DMA,
    ],
    compiler_params=pltpu.CompilerParams(needs_layout_passes=False),
)
def _sc_gcn(IembT, UembT, IembP, UembP, oldUT, oldIT, oldU1T, pA, vA, pB, vB,
            user, item_i, item_j, u_sample, i_sample,
            o_Ie_ii, o_Ie_ij, o_Ie_is,
            o_g1u_user, o_g1u_us,
            o_g2i_ii, o_g2i_ij, o_g2i_is,
            o_g3u_user, o_g3u_us,
            o_Ue_user, o_Ue_us,
            o_g1i_ii, o_g1i_ij, o_g1i_is,
            o_g2u_user, o_g2u_us,
            o_g3i_ii, o_g3i_ij, o_g3i_is,
            o_oU_us, o_oI_is, o_oU1_us,
            pk, acc, tmp, x_user, x_us, x_ii, x_ij, x_is,
            ep0, ep1, ev0, ev1, s0, s1):
    w = lax.axis_index("s") * 2 + lax.axis_index("c")
    tb = w * _SLICE
    pbase = w * _PSLICE
    to = w * _OUTW

    pltpu.sync_copy(user, x_user)
    pltpu.sync_copy(u_sample, x_us)
    pltpu.sync_copy(item_i, x_ii)
    pltpu.sync_copy(item_j, x_ij)
    pltpu.sync_copy(i_sample, x_is)

    aslc = [acc.at[pl.ds(f * _U, _U)] for f in range(_FPT)]
    pslc = [pk.at[pl.ds(j * _U, _U)] for j in range(_NPAIR)]

    def load_f32_slice(src_hbm):
        pltpu.sync_copy(src_hbm.at[pl.ds(tb, _SLICE)], acc)

    def load_packed_slice(src_hbm):
        pltpu.sync_copy(src_hbm.at[pl.ds(pbase, _PSLICE)], pk)

    def zero_acc():
        z = jnp.zeros((16,), jnp.float32)

        @plsc.parallel_loop(0, _SLICE // 16, unroll=8)
        def _(i):
            acc[pl.ds(i * 16, 16)] = z

    def repack():
        @plsc.parallel_loop(0, _U // 16, unroll=4)
        def _(i):
            for j in range(_NPAIR):
                a = acc[pl.ds((2 * j) * _U + i * 16, 16)]
                b = acc[pl.ds((2 * j + 1) * _U + i * 16, 16)]
                p = plsc.pack(a, b, format=plsc.PackFormat.INTERLEAVED)
                pk[pl.ds(j * _U + i * 16, 16)] = plsc.bitcast(p, jnp.int32)

    def gather_dump(idxref, out_hbm):
        @plsc.parallel_loop(0, _B // 16, unroll=4)
        def _(i):
            ix = idxref[pl.ds(i * 16, 16)]
            for f in range(_FPT):
                g = plsc.load_gather(aslc[f], [ix])
                tmp[pl.ds(f * _B + i * 16, 16)] = g

        pltpu.sync_copy(tmp, out_hbm.at[pl.ds(to, _OUTW)])

    def edge_pass(phbm, vhbm):
        pltpu.async_copy(phbm.at[pl.ds(0, _EC)], ep0, s0)
        pltpu.async_copy(vhbm.at[pl.ds(0, _EC)], ev0, s0)
        zero_acc()

        def consume(epb, evb):
            @plsc.parallel_loop(0, _VPC, unroll=5)
            def _(i):
                pkv = epb[pl.ds(i * 16, 16)]
                vv = evb[pl.ds(i * 16, 16)]
                src = jnp.bitwise_and(pkv, 0x3FFF)
                dst = jnp.right_shift(pkv, 14)
                for j in range(_NPAIR):
                    gw = plsc.load_gather(pslc[j], [src])
                    ga, gb = plsc.unpack(
                        plsc.bitcast(gw, jnp.bfloat16),
                        format=plsc.PackFormat.INTERLEAVED)
                    plsc.addupdate_scatter(aslc[2 * j], [dst], ga * vv)
                    plsc.addupdate_scatter(aslc[2 * j + 1], [dst], gb * vv)

        def drain(epb, evb, sem):
            pltpu.make_async_copy(phbm.at[pl.ds(0, _EC)], epb, sem).wait()
            pltpu.make_async_copy(vhbm.at[pl.ds(0, _EC)], evb, sem).wait()

        def outer(k, carry):
            c1 = 2 * k + 1
            pltpu.async_copy(phbm.at[pl.ds(c1 * _EC, _EC)], ep1, s1)
            pltpu.async_copy(vhbm.at[pl.ds(c1 * _EC, _EC)], ev1, s1)
            drain(ep0, ev0, s0)
            consume(ep0, ev0)

            @pl.when(k < _NCH // 2 - 1)
            def _():
                c2 = 2 * k + 2
                pltpu.async_copy(phbm.at[pl.ds(c2 * _EC, _EC)], ep0, s0)
                pltpu.async_copy(vhbm.at[pl.ds(c2 * _EC, _EC)], ev0, s0)

            drain(ep1, ev1, s1)
            consume(ep1, ev1)
            return carry

        lax.fori_loop(0, _NCH // 2, outer, 0)

    load_f32_slice(IembT)
    gather_dump(x_ii, o_Ie_ii)
    gather_dump(x_ij, o_Ie_ij)
    gather_dump(x_is, o_Ie_is)
    load_packed_slice(IembP)

    edge_pass(pA, vA)
    gather_dump(x_user, o_g1u_user)
    gather_dump(x_us, o_g1u_us)
    repack()

    edge_pass(pB, vB)
    gather_dump(x_ii, o_g2i_ii)
    gather_dump(x_ij, o_g2i_ij)
    gather_dump(x_is, o_g2i_is)
    repack()

    edge_pass(pA, vA)
    gather_dump(x_user, o_g3u_user)
    gather_dump(x_us, o_g3u_us)

    load_f32_slice(UembT)
    gather_dump(x_user, o_Ue_user)
    gather_dump(x_us, o_Ue_us)
    load_packed_slice(UembP)

    edge_pass(pB, vB)
    gather_dump(x_ii, o_g1i_ii)
    gather_dump(x_ij, o_g1i_ij)
    gather_dump(x_is, o_g1i_is)
    repack()

    edge_pass(pA, vA)
    gather_dump(x_user, o_g2u_user)
    gather_dump(x_us, o_g2u_us)
    repack()

    edge_pass(pB, vB)
    gather_dump(x_ii, o_g3i_ii)
    gather_dump(x_ij, o_g3i_ij)
    gather_dump(x_is, o_g3i_is)

    load_f32_slice(oldUT)
    gather_dump(x_us, o_oU_us)
    load_f32_slice(oldIT)
    gather_dump(x_is, o_oI_is)
    load_f32_slice(oldU1T)
    gather_dump(x_us, o_oU1_us)


_BT = 512
_NSTEP = _B // _BT


def _softplus(x):
    return jnp.maximum(x, 0.0) + jnp.log(1.0 + jnp.exp(-jnp.abs(x)))


def _leaky(x):
    return jnp.where(x >= 0.0, x, 0.01 * x)


def _tc_body(Ie_ii, Ie_ij, Ie_is, g1u_user, g1u_us, g2i_ii, g2i_ij, g2i_is,
             g3u_user, g3u_us, Ue_user, Ue_us, g1i_ii, g1i_ij, g1i_is,
             g2u_user, g2u_us, g3i_ii, g3i_ij, g3i_is, oU_us, oI_is, oU1_us,
             pw0, pb0, pw1, pb1, pw2, pb2, sw0, sb0, sw1t, sb1,
             out, acc):
    step = pl.program_id(0)

    @pl.when(step == 0)
    def _():
        acc[0] = 0.0
        acc[1] = 0.0
        acc[2] = 0.0
        acc[3] = 0.0

    c1, c2, c3 = 0.5, 1.0 / 3.0, 0.25
    u_user = Ue_user[...] + c1 * g1u_user[...] + c2 * g2u_user[...] + c3 * g3u_user[...]
    u_us = Ue_us[...] + c1 * g1u_us[...] + c2 * g2u_us[...] + c3 * g3u_us[...]
    i_ii = Ie_ii[...] + c1 * g1i_ii[...] + c2 * g2i_ii[...] + c3 * g3i_ii[...]
    i_ij = Ie_ij[...] + c1 * g1i_ij[...] + c2 * g2i_ij[...] + c3 * g3i_ij[...]
    i_is = Ie_is[...] + c1 * g1i_is[...] + c2 * g2i_is[...] + c3 * g3i_is[...]

    pred_i = jnp.sum(u_user * i_ii, axis=0, keepdims=True)
    pred_j = jnp.sum(u_user * i_ij, axis=0, keepdims=True)
    s_bpr = jnp.sum(_softplus(pred_j - pred_i))
    s_reg = jnp.sum(u_user * u_user + i_ii * i_ii + i_ij * i_ij)

    def mlp_pref(xT):
        dn = (((0,), (0,)), ((), ()))
        h = _leaky(lax.dot_general(pw0[...], xT, dn,
                                   preferred_element_type=jnp.float32) + pb0[...])
        h = _leaky(lax.dot_general(pw1[...], h, dn,
                                   preferred_element_type=jnp.float32) + pb1[...])
        p = lax.dot_general(pw2[...], h, dn,
                            preferred_element_type=jnp.float32) + pb2[...]
        p = p - jnp.max(p, axis=0, keepdims=True)
        e = jnp.exp(p)
        return e / jnp.sum(e, axis=0, keepdims=True)

    p_new = mlp_pref(g1u_us[...])
    p_old = mlp_pref(oU1_us[...])
    s = (p_old - p_new) * (p_old - p_new)
    dn = (((0,), (0,)), ((), ()))
    hs = jnp.maximum(lax.dot_general(sw0[...], s, dn,
                                     preferred_element_type=jnp.float32) + sb0[...],
                     0.0)
    wu = _softplus(lax.dot_general(sw1t[...], hs, (((1,), (0,)), ((), ())),
                                   preferred_element_type=jnp.float32) + sb1[...])
    s_wu = jnp.sum(wu)

    diff = (jnp.sum(u_us * i_is, axis=0, keepdims=True)
            - jnp.sum(oU_us[...] * oI_is[...], axis=0, keepdims=True))
    s_d2 = jnp.sum(diff * diff)

    acc[0] += s_bpr
    acc[1] += s_reg
    acc[2] += s_wu
    acc[3] += s_d2

    @pl.when(step == _NSTEP - 1)
    def _():
        inv_b = 1.0 / _B
        loss_bpr = acc[0] * inv_b + 1e-4 * acc[1] * inv_b
        loss_kd = (acc[2] * inv_b) * (acc[3] * inv_b)
        out[0, 0] = loss_bpr + 0.01 * loss_kd


def _tc_loss(gathered, pw0, pb0, pw1, pb1, pw2, pb2, sw0, sb0, sw1, sb1):
    col = pl.BlockSpec((_FDIM, _BT), lambda i: (0, i))

    def full(shape):
        return pl.BlockSpec(shape, lambda i: tuple(0 for _ in shape))

    in_specs = [col] * 23 + [
        full((_FDIM, _FDIM)), full((_FDIM, 1)),
        full((_FDIM, _FDIM)), full((_FDIM, 1)),
        full((_FDIM, _NPREF)), full((_NPREF, 1)),
        full((_NPREF, _FDIM)), full((_FDIM, 1)),
        full((1, _FDIM)), full((1, 1)),
    ]
    out = pl.pallas_call(
        _tc_body,
        grid=(_NSTEP,),
        in_specs=in_specs,
        out_specs=pl.BlockSpec(memory_space=pltpu.SMEM),
        out_shape=jax.ShapeDtypeStruct((1, 1), jnp.float32),
        scratch_shapes=[pltpu.SMEM((4,), jnp.float32)],
        compiler_params=pltpu.CompilerParams(
            dimension_semantics=("arbitrary",)),
    )(*gathered,
      pw0, pb0.reshape(_FDIM, 1), pw1, pb1.reshape(_FDIM, 1),
      pw2, pb2.reshape(_NPREF, 1), sw0, sb0.reshape(_FDIM, 1),
      sw1.reshape(1, _FDIM), sb1.reshape(1, 1))
    return out[0, 0]


def _pack_pairs(tT):
    pairs = tT.reshape(_FDIM // 2, 2, -1)
    lo = lax.bitcast_convert_type(
        pairs[:, 0, :].astype(jnp.bfloat16), jnp.uint16).astype(jnp.uint32)
    hi = lax.bitcast_convert_type(
        pairs[:, 1, :].astype(jnp.bfloat16), jnp.uint16).astype(jnp.uint32)
    return lax.bitcast_convert_type(lo | (hi << 16), jnp.int32).reshape(-1)


def kernel(user, item_i, item_j, ui_rows, ui_cols, ui_vals, iu_vals,
           embed_user_weight, embed_item_weight, u_sample, i_sample,
           old_U_emb, old_I_emb, old_User1,
           pw0, pb0, pw1, pb1, pw2, pb2, sw0, sb0, sw1, sb1):
    IembT = embed_item_weight.T
    UembT = embed_user_weight.T
    oldUT = old_U_emb.T.reshape(-1)
    oldIT = old_I_emb.T.reshape(-1)
    oldU1T = old_User1.T.reshape(-1)
    rows = ui_rows.astype(jnp.int32)
    cols = ui_cols.astype(jnp.int32)
    pA = jnp.left_shift(rows, 14) | cols
    pB = jnp.left_shift(cols, 14) | rows

    gathered = _sc_gcn(IembT.reshape(-1), UembT.reshape(-1),
                       _pack_pairs(IembT), _pack_pairs(UembT),
                       oldUT, oldIT, oldU1T,
                       pA, ui_vals, pB, iu_vals,
                       user.astype(jnp.int32), item_i.astype(jnp.int32),
                       item_j.astype(jnp.int32), u_sample.astype(jnp.int32),
                       i_sample.astype(jnp.int32))
    gathered = [g.reshape(_FDIM, _B) for g in gathered]
    return _tc_loss(gathered, pw0, pb0, pw1, pb1, pw2, pb2,
                    sw0, sb0, sw1, sb1)


# speedup vs baseline: 1.0140x; 1.0056x over previous
import functools

import jax
import jax.numpy as jnp
from jax import lax
from jax.experimental import pallas as pl
from jax.experimental.pallas import tpu as pltpu
from jax.experimental.pallas import tpu_sc as plsc

_U = 10000
_I = 10000
_FDIM = 128
_NNZ = 320000
_NPREF = 32
_B = 4096

_NTILES = 32
_FPT = _FDIM // _NTILES
_NPAIR = _FPT // 2
_SLICE = _FPT * _U
_PSLICE = _NPAIR * _U
_OUTW = _FPT * _B
_EC = 4000
_NCH = _NNZ // _EC
_VPC = _EC // 16

_mesh = plsc.VectorSubcoreMesh(core_axis_name="c", subcore_axis_name="s")
_gout = jax.ShapeDtypeStruct((_NTILES * _OUTW,), jnp.float32)


@functools.partial(
    pl.kernel,
    out_type=tuple([_gout] * 23),
    mesh=_mesh,
    scratch_types=[
        pltpu.VMEM((_PSLICE,), jnp.int32),
        pltpu.VMEM((_SLICE,), jnp.float32),
        pltpu.VMEM((_OUTW,), jnp.float32),
        pltpu.VMEM((_B,), jnp.int32),
        pltpu.VMEM((_B,), jnp.int32),
        pltpu.VMEM((_B,), jnp.int32),
        pltpu.VMEM((_B,), jnp.int32),
        pltpu.VMEM((_B,), jnp.int32),
        pltpu.VMEM((_EC,), jnp.int32),
        pltpu.VMEM((_EC,), jnp.int32),
        pltpu.VMEM((_EC,), jnp.float32),
        pltpu.VMEM((_EC,), jnp.float32),
        pltpu.SemaphoreType.DMA,
        pltpu.SemaphoreType.DMA,
    ],
    compiler_params=pltpu.CompilerParams(needs_layout_passes=False),
)
def _sc_gcn(IembT, UembT, IembP, UembP, oldUT, oldIT, oldU1T, pA, vA, pB, vB,
            user, item_i, item_j, u_sample, i_sample,
            o_Ie_ii, o_Ie_ij, o_Ie_is,
            o_g1u_user, o_g1u_us,
            o_g2i_ii, o_g2i_ij, o_g2i_is,
            o_g3u_user, o_g3u_us,
            o_Ue_user, o_Ue_us,
            o_g1i_ii, o_g1i_ij, o_g1i_is,
            o_g2u_user, o_g2u_us,
            o_g3i_ii, o_g3i_ij, o_g3i_is,
            o_oU_us, o_oI_is, o_oU1_us,
            pk, acc, tmp, x_user, x_us, x_ii, x_ij, x_is,
            ep0, ep1, ev0, ev1, s0, s1):
    w = lax.axis_index("s") * 2 + lax.axis_index("c")
    tb = w * _SLICE
    pbase = w * _PSLICE
    to = w * _OUTW

    pltpu.sync_copy(user, x_user)
    pltpu.sync_copy(u_sample, x_us)
    pltpu.sync_copy(item_i, x_ii)
    pltpu.sync_copy(item_j, x_ij)
    pltpu.sync_copy(i_sample, x_is)

    aslc = [acc.at[pl.ds(f * _U, _U)] for f in range(_FPT)]
    pslc = [pk.at[pl.ds(j * _U, _U)] for j in range(_NPAIR)]

    def load_f32_slice(src_hbm):
        pltpu.sync_copy(src_hbm.at[pl.ds(tb, _SLICE)], acc)

    def load_packed_slice(src_hbm):
        pltpu.sync_copy(src_hbm.at[pl.ds(pbase, _PSLICE)], pk)

    def zero_acc():
        z = jnp.zeros((16,), jnp.float32)

        @plsc.parallel_loop(0, _SLICE // 16, unroll=8)
        def _(i):
            acc[pl.ds(i * 16, 16)] = z

    def repack():
        @plsc.parallel_loop(0, _U // 16, unroll=4)
        def _(i):
            for j in range(_NPAIR):
                a = acc[pl.ds((2 * j) * _U + i * 16, 16)]
                b = acc[pl.ds((2 * j + 1) * _U + i * 16, 16)]
                p = plsc.pack(a, b, format=plsc.PackFormat.INTERLEAVED)
                pk[pl.ds(j * _U + i * 16, 16)] = plsc.bitcast(p, jnp.int32)

    def gather_dump(idxref, out_hbm):
        @plsc.parallel_loop(0, _B // 16, unroll=4)
        def _(i):
            ix = idxref[pl.ds(i * 16, 16)]
            for f in range(_FPT):
                g = plsc.load_gather(aslc[f], [ix])
                tmp[pl.ds(f * _B + i * 16, 16)] = g

        pltpu.sync_copy(tmp, out_hbm.at[pl.ds(to, _OUTW)])

    def edge_pass(phbm, vhbm):
        pltpu.async_copy(phbm.at[pl.ds(0, _EC)], ep0, s0)
        pltpu.async_copy(vhbm.at[pl.ds(0, _EC)], ev0, s0)
        zero_acc()

        def consume(epb, evb):
            @plsc.parallel_loop(0, _VPC, unroll=5)
            def _(i):
                pkv = epb[pl.ds(i * 16, 16)]
                vv = evb[pl.ds(i * 16, 16)]
                src = jnp.bitwise_and(pkv, 0x3FFF)
                dst = jnp.right_shift(pkv, 14)
                for j in range(_NPAIR):
                    gw = plsc.load_gather(pslc[j], [src])
                    ga, gb = plsc.unpack(
                        plsc.bitcast(gw, jnp.bfloat16),
                        format=plsc.PackFormat.INTERLEAVED)
                    plsc.addupdate_scatter(aslc[2 * j], [dst], ga * vv)
                    plsc.addupdate_scatter(aslc[2 * j + 1], [dst], gb * vv)

        def drain(epb, evb, sem):
            pltpu.make_async_copy(phbm.at[pl.ds(0, _EC)], epb, sem).wait()
            pltpu.make_async_copy(vhbm.at[pl.ds(0, _EC)], evb, sem).wait()

        def outer(k, carry):
            c1 = 2 * k + 1
            pltpu.async_copy(phbm.at[pl.ds(c1 * _EC, _EC)], ep1, s1)
            pltpu.async_copy(vhbm.at[pl.ds(c1 * _EC, _EC)], ev1, s1)
            drain(ep0, ev0, s0)
            consume(ep0, ev0)

            @pl.when(k < _NCH // 2 - 1)
            def _():
                c2 = 2 * k + 2
                pltpu.async_copy(phbm.at[pl.ds(c2 * _EC, _EC)], ep0, s0)
                pltpu.async_copy(vhbm.at[pl.ds(c2 * _EC, _EC)], ev0, s0)

            drain(ep1, ev1, s1)
            consume(ep1, ev1)
            return carry

        lax.fori_loop(0, _NCH // 2, outer, 0)

    load_f32_slice(IembT)
    gather_dump(x_ii, o_Ie_ii)
    gather_dump(x_ij, o_Ie_ij)
    gather_dump(x_is, o_Ie_is)
    load_packed_slice(IembP)

    edge_pass(pA, vA)
    gather_dump(x_user, o_g1u_user)
    gather_dump(x_us, o_g1u_us)
    repack()

    edge_pass(pB, vB)
    gather_dump(x_ii, o_g2i_ii)
    gather_dump(x_ij, o_g2i_ij)
    gather_dump(x_is, o_g2i_is)
    repack()

    edge_pass(pA, vA)
    gather_dump(x_user, o_g3u_user)
    gather_dump(x_us, o_g3u_us)

    load_f32_slice(UembT)
    gather_dump(x_user, o_Ue_user)
    gather_dump(x_us, o_Ue_us)
    load_packed_slice(UembP)

    edge_pass(pB, vB)
    gather_dump(x_ii, o_g1i_ii)
    gather_dump(x_ij, o_g1i_ij)
    gather_dump(x_is, o_g1i_is)
    repack()

    edge_pass(pA, vA)
    gather_dump(x_user, o_g2u_user)
    gather_dump(x_us, o_g2u_us)
    repack()

    edge_pass(pB, vB)
    gather_dump(x_ii, o_g3i_ii)
    gather_dump(x_ij, o_g3i_ij)
    gather_dump(x_is, o_g3i_is)

    load_f32_slice(oldUT)
    gather_dump(x_us, o_oU_us)
    load_f32_slice(oldIT)
    gather_dump(x_is, o_oI_is)
    load_f32_slice(oldU1T)
    gather_dump(x_us, o_oU1_us)


_BT = 512
_NSTEP = _B // _BT


def _softplus(x):
    return jnp.maximum(x, 0.0) + jnp.log(1.0 + jnp.exp(-jnp.abs(x)))


def _leaky(x):
    return jnp.where(x >= 0.0, x, 0.01 * x)


def _tc_body(Ie_ii, Ie_ij, Ie_is, g1u_user, g1u_us, g2i_ii, g2i_ij, g2i_is,
             g3u_user, g3u_us, Ue_user, Ue_us, g1i_ii, g1i_ij, g1i_is,
             g2u_user, g2u_us, g3i_ii, g3i_ij, g3i_is, oU_us, oI_is, oU1_us,
             pw0, pb0, pw1, pb1, pw2, pb2, sw0, sb0, sw1t, sb1,
             out, acc):
    step = pl.program_id(0)

    @pl.when(step == 0)
    def _():
        acc[0] = 0.0
        acc[1] = 0.0
        acc[2] = 0.0
        acc[3] = 0.0

    c1, c2, c3 = 0.5, 1.0 / 3.0, 0.25
    u_user = Ue_user[...] + c1 * g1u_user[...] + c2 * g2u_user[...] + c3 * g3u_user[...]
    u_us = Ue_us[...] + c1 * g1u_us[...] + c2 * g2u_us[...] + c3 * g3u_us[...]
    i_ii = Ie_ii[...] + c1 * g1i_ii[...] + c2 * g2i_ii[...] + c3 * g3i_ii[...]
    i_ij = Ie_ij[...] + c1 * g1i_ij[...] + c2 * g2i_ij[...] + c3 * g3i_ij[...]
    i_is = Ie_is[...] + c1 * g1i_is[...] + c2 * g2i_is[...] + c3 * g3i_is[...]

    pred_i = jnp.sum(u_user * i_ii, axis=0, keepdims=True)
    pred_j = jnp.sum(u_user * i_ij, axis=0, keepdims=True)
    s_bpr = jnp.sum(_softplus(pred_j - pred_i))
    s_reg = jnp.sum(u_user * u_user + i_ii * i_ii + i_ij * i_ij)

    def mlp_pref(xT):
        dn = (((0,), (0,)), ((), ()))
        h = _leaky(lax.dot_general(pw0[...], xT, dn,
                                   preferred_element_type=jnp.float32) + pb0[...])
        h = _leaky(lax.dot_general(pw1[...], h, dn,
                                   preferred_element_type=jnp.float32) + pb1[...])
        p = lax.dot_general(pw2[...], h, dn,
                            preferred_element_type=jnp.float32) + pb2[...]
        p = p - jnp.max(p, axis=0, keepdims=True)
        e = jnp.exp(p)
        return e / jnp.sum(e, axis=0, keepdims=True)

    p_new = mlp_pref(g1u_us[...])
    p_old = mlp_pref(oU1_us[...])
    s = (p_old - p_new) * (p_old - p_new)
    dn = (((0,), (0,)), ((), ()))
    hs = jnp.maximum(lax.dot_general(sw0[...], s, dn,
                                     preferred_element_type=jnp.float32) + sb0[...],
                     0.0)
    wu = _softplus(lax.dot_general(sw1t[...], hs, (((1,), (0,)), ((), ())),
                                   preferred_element_type=jnp.float32) + sb1[...])
    s_wu = jnp.sum(wu)

    diff = (jnp.sum(u_us * i_is, axis=0, keepdims=True)
            - jnp.sum(oU_us[...] * oI_is[...], axis=0, keepdims=True))
    s_d2 = jnp.sum(diff * diff)

    acc[0] += s_bpr
    acc[1] += s_reg
    acc[2] += s_wu
    acc[3] += s_d2

    @pl.when(step == _NSTEP - 1)
    def _():
        inv_b = 1.0 / _B
        loss_bpr = acc[0] * inv_b + 1e-4 * acc[1] * inv_b
        loss_kd = (acc[2] * inv_b) * (acc[3] * inv_b)
        out[0, 0] = loss_bpr + 0.01 * loss_kd


def _tc_loss(gathered, pw0, pb0, pw1, pb1, pw2, pb2, sw0, sb0, sw1, sb1):
    col = pl.BlockSpec((_FDIM, _BT), lambda i: (0, i))

    def full(shape):
        return pl.BlockSpec(shape, lambda i: tuple(0 for _ in shape))

    in_specs = [col] * 23 + [
        full((_FDIM, _FDIM)), full((_FDIM, 1)),
        full((_FDIM, _FDIM)), full((_FDIM, 1)),
        full((_FDIM, _NPREF)), full((_NPREF, 1)),
        full((_NPREF, _FDIM)), full((_FDIM, 1)),
        full((1, _FDIM)), full((1, 1)),
    ]
    out = pl.pallas_call(
        _tc_body,
        grid=(_NSTEP,),
        in_specs=in_specs,
        out_specs=pl.BlockSpec(memory_space=pltpu.SMEM),
        out_shape=jax.ShapeDtypeStruct((1, 1), jnp.float32),
        scratch_shapes=[pltpu.SMEM((4,), jnp.float32)],
        compiler_params=pltpu.CompilerParams(
            dimension_semantics=("arbitrary",)),
    )(*gathered,
      pw0, pb0.reshape(_FDIM, 1), pw1, pb1.reshape(_FDIM, 1),
      pw2, pb2.reshape(_NPREF, 1), sw0, sb0.reshape(_FDIM, 1),
      sw1.reshape(1, _FDIM), sb1.reshape(1, 1))
    return out[0, 0]


def _pack_pairs(tT):
    pairs = tT.reshape(_FDIM // 2, 2, -1)
    lo = lax.bitcast_convert_type(
        pairs[:, 0, :].astype(jnp.bfloat16), jnp.uint16).astype(jnp.uint32)
    hi = lax.bitcast_convert_type(
        pairs[:, 1, :].astype(jnp.bfloat16), jnp.uint16).astype(jnp.uint32)
    return lax.bitcast_convert_type(lo | (hi << 16), jnp.int32).reshape(-1)


def kernel(user, item_i, item_j, ui_rows, ui_cols, ui_vals, iu_vals,
           embed_user_weight, embed_item_weight, u_sample, i_sample,
           old_U_emb, old_I_emb, old_User1,
           pw0, pb0, pw1, pb1, pw2, pb2, sw0, sb0, sw1, sb1):
    IembT = embed_item_weight.T
    UembT = embed_user_weight.T
    oldUT = old_U_emb.T.reshape(-1)
    oldIT = old_I_emb.T.reshape(-1)
    oldU1T = old_User1.T.reshape(-1)
    rows = ui_rows.astype(jnp.int32)
    cols = ui_cols.astype(jnp.int32)
    pA = jnp.left_shift(rows, 14) | cols
    pB = jnp.left_shift(cols, 14) | rows

    gathered = _sc_gcn(IembT.reshape(-1), UembT.reshape(-1),
                       _pack_pairs(IembT), _pack_pairs(UembT),
                       oldUT, oldIT, oldU1T,
                       pA, ui_vals, pB, iu_vals,
                       user.astype(jnp.int32), item_i.astype(jnp.int32),
                       item_j.astype(jnp.int32), u_sample.astype(jnp.int32),
                       i_sample.astype(jnp.int32))
    gathered = [g.reshape(_FDIM, _B) for g in gathered]
    return _tc_loss(gathered, pw0, pb0, pw1, pb1, pw2, pb2,
                    sw0, sb0, sw1, sb1)
